# baseline (speedup 1.0000x reference)
import functools
import math

import jax
import jax.numpy as jnp
from jax import lax
from jax.experimental import pallas as pl
from jax.experimental.pallas import tpu as pltpu

_INV_SQRT2 = 1.0 / math.sqrt(2.0)


def _gelu_exact(x):
    return 0.5 * x * (1.0 + lax.erf(x * _INV_SQRT2))


def _ffn_kernel(x_hbm, w1_hbm, b1_ref, w2_hbm, b2_ref, o_hbm,
                xin, xb, w1l, w2l, w1b, w2b, hbuf, obuf,
                sx, sw1, sw2, so, *, nk, nj, th, tmj, rows_core):
    i = pl.program_id(0)
    r0 = i * rows_core

    def x_copy(j, slot):
        return pltpu.make_async_copy(
            x_hbm.at[pl.ds(r0 + j * tmj, tmj), :], xin.at[slot], sx.at[slot])

    def w1_copy(k, slot):
        return pltpu.make_async_copy(
            w1_hbm.at[:, pl.ds(k * th, th)], w1l.at[slot], sw1.at[slot])

    def w2_copy(k, slot):
        return pltpu.make_async_copy(
            w2_hbm.at[pl.ds(k * th, th), :], w2l.at[slot], sw2.at[slot])

    def o_copy(j):
        return pltpu.make_async_copy(
            obuf.at[0], o_hbm.at[pl.ds(r0 + j * tmj, tmj), :], so.at[0])

    x_copy(0, 0).start()
    if nj > 1:
        x_copy(1, 1).start()
    w1_copy(0, 0).start()
    w2_copy(0, 0).start()
    if nk > 1:
        w1_copy(1, 1).start()
        w2_copy(1, 1).start()

    b2f = b2_ref[...].astype(jnp.float32)

    for k in range(nk):
        sl = k % 2
        hcols = pl.ds(k * th, th)
        w1_copy(k, sl).wait()
        w1b[:, hcols] = w1l[sl].astype(jnp.bfloat16)
        w2_copy(k, sl).wait()
        w2b[hcols, :] = w2l[sl].astype(jnp.bfloat16)
        if k + 2 < nk:
            w1_copy(k + 2, sl).start()
            w2_copy(k + 2, sl).start()
        b1k = b1_ref[:, hcols].astype(jnp.float32)
        w1c = w1b[:, hcols]
        for j in range(nj):
            if k == 0:
                x_copy(j, j % 2).wait()
                if j + 2 < nj:
                    x_copy(j + 2, j % 2).start()
                xb[pl.ds(j * tmj, tmj), :] = xin[j % 2].astype(jnp.bfloat16)
            h = jnp.dot(xb[pl.ds(j * tmj, tmj), :], w1c,
                        preferred_element_type=jnp.float32)
            hbuf[pl.ds(j * tmj, tmj), hcols] = (
                _gelu_exact(h + b1k).astype(jnp.bfloat16))

    for j in range(nj):
        o = jnp.dot(hbuf[pl.ds(j * tmj, tmj), :], w2b[...],
                    preferred_element_type=jnp.float32) + b2f
        if j >= 1:
            o_copy(j - 1).wait()
        obuf[0] = o
        o_copy(j).start()

    o_copy(nj - 1).wait()


def kernel(x, w1, b1, w2, b2):
    batch, seq, dim = x.shape
    hidden = w1.shape[1]
    M = batch * seq
    x2d = x.reshape(M, dim)

    b1r = b1.reshape(1, hidden).astype(jnp.float32)
    b2r = b2.reshape(1, dim).astype(jnp.float32)

    nj = 4
    tmj = 512
    Mp = -(-M // (2 * nj * tmj)) * (2 * nj * tmj)
    if Mp != M:
        x2d = jnp.pad(x2d, ((0, Mp - M), (0, 0)))
    rows_core = Mp // 2
    tmj = rows_core // nj

    th = 256 if hidden % 256 == 0 else hidden
    nk = hidden // th

    cost = pl.CostEstimate(
        flops=int(4 * Mp * dim * hidden),
        transcendentals=int(Mp * hidden),
        bytes_accessed=int(4 * Mp * dim * 2 + 2 * (dim * hidden * 4)),
    )

    out2d = pl.pallas_call(
        functools.partial(_ffn_kernel, nk=nk, nj=nj, th=th, tmj=tmj,
                          rows_core=rows_core),
        out_shape=jax.ShapeDtypeStruct((Mp, dim), x.dtype),
        grid=(2,),
        in_specs=[
            pl.BlockSpec(memory_space=pl.ANY),
            pl.BlockSpec(memory_space=pl.ANY),
            pl.BlockSpec((1, hidden), lambda i: (0, 0)),
            pl.BlockSpec(memory_space=pl.ANY),
            pl.BlockSpec((1, dim), lambda i: (0, 0)),
        ],
        out_specs=pl.BlockSpec(memory_space=pl.ANY),
        scratch_shapes=[
            pltpu.VMEM((2, tmj, dim), jnp.float32),
            pltpu.VMEM((rows_core, dim), jnp.bfloat16),
            pltpu.VMEM((2, dim, th), jnp.float32),
            pltpu.VMEM((2, th, dim), jnp.float32),
            pltpu.VMEM((dim, hidden), jnp.bfloat16),
            pltpu.VMEM((hidden, dim), jnp.bfloat16),
            pltpu.VMEM((rows_core, hidden), jnp.bfloat16),
            pltpu.VMEM((1, tmj, dim), jnp.float32),
            pltpu.SemaphoreType.DMA((2,)),
            pltpu.SemaphoreType.DMA((2,)),
            pltpu.SemaphoreType.DMA((2,)),
            pltpu.SemaphoreType.DMA((2,)),
        ],
        compiler_params=pltpu.CompilerParams(
            dimension_semantics=("parallel",),
            vmem_limit_bytes=61 * 1024 * 1024,
        ),
        cost_estimate=cost,
    )(x2d, w1, b1r, w2, b2r)

    if Mp != M:
        out2d = out2d[:M]
    return out2d.reshape(batch, seq, dim)


# speedup vs baseline: 1.0085x; 1.0085x over previous
import math

import jax
import jax.numpy as jnp
from jax import lax
from jax.experimental import pallas as pl
from jax.experimental.pallas import tpu as pltpu

_INV_SQRT2 = 1.0 / math.sqrt(2.0)


def _gelu_exact(x):
    return 0.5 * x * (1.0 + lax.erf(x * _INV_SQRT2))


def _ffn_kernel(x_ref, w1_ref, b1_ref, w2_ref, b2_ref, o_ref):
    xb = x_ref[...].astype(jnp.bfloat16)
    h = jnp.dot(xb, w1_ref[...], preferred_element_type=jnp.float32)
    h = _gelu_exact(h + b1_ref[...].astype(jnp.float32))
    o = jnp.dot(h.astype(jnp.bfloat16), w2_ref[...],
                preferred_element_type=jnp.float32)
    o_ref[...] = (o + b2_ref[...].astype(jnp.float32)).astype(o_ref.dtype)


def kernel(x, w1, b1, w2, b2):
    batch, seq, dim = x.shape
    hidden = w1.shape[1]
    M = batch * seq
    x2d = x.reshape(M, dim)

    w1b = w1.astype(jnp.bfloat16)
    w2b = w2.astype(jnp.bfloat16)
    b1r = b1.reshape(1, hidden).astype(jnp.float32)
    b2r = b2.reshape(1, dim).astype(jnp.float32)

    TM = 512
    Mp = -(-M // (2 * TM)) * (2 * TM)
    if Mp != M:
        x2d = jnp.pad(x2d, ((0, Mp - M), (0, 0)))

    cost = pl.CostEstimate(
        flops=int(4 * Mp * dim * hidden),
        transcendentals=int(Mp * hidden),
        bytes_accessed=int(4 * Mp * dim * 2 + 2 * (dim * hidden * 2)),
    )

    out2d = pl.pallas_call(
        _ffn_kernel,
        out_shape=jax.ShapeDtypeStruct((Mp, dim), x.dtype),
        grid=(Mp // TM,),
        in_specs=[
            pl.BlockSpec((TM, dim), lambda i: (i, 0)),
            pl.BlockSpec((dim, hidden), lambda i: (0, 0)),
            pl.BlockSpec((1, hidden), lambda i: (0, 0)),
            pl.BlockSpec((hidden, dim), lambda i: (0, 0)),
            pl.BlockSpec((1, dim), lambda i: (0, 0)),
        ],
        out_specs=pl.BlockSpec((TM, dim), lambda i: (i, 0)),
        compiler_params=pltpu.CompilerParams(
            dimension_semantics=("parallel",),
            vmem_limit_bytes=61 * 1024 * 1024,
        ),
        cost_estimate=cost,
    )(x2d, w1b, b1r, w2b, b2r)

    if Mp != M:
        out2d = out2d[:M]
    return out2d.reshape(batch, seq, dim)


# speedup vs baseline: 1.1571x; 1.1474x over previous
import functools
import math

import jax
import jax.numpy as jnp
from jax import lax
from jax.experimental import pallas as pl
from jax.experimental.pallas import tpu as pltpu

_INV_SQRT2 = 1.0 / math.sqrt(2.0)


def _gelu_exact(x):
    return 0.5 * x * (1.0 + lax.erf(x * _INV_SQRT2))


def _ffn_kernel(x_ref, w1_ref, b1_ref, w2_ref, b2_ref, o_ref, *, th):
    xb = x_ref[...].astype(jnp.bfloat16)
    n_h = w1_ref.shape[1] // th
    acc = jnp.broadcast_to(b2_ref[...].astype(jnp.float32), o_ref.shape)
    for c in range(n_h):
        w1c = w1_ref[:, c * th:(c + 1) * th].astype(jnp.bfloat16)
        h = jnp.dot(xb, w1c, preferred_element_type=jnp.float32)
        h = _gelu_exact(h + b1_ref[:, c * th:(c + 1) * th].astype(jnp.float32))
        w2c = w2_ref[c * th:(c + 1) * th, :].astype(jnp.bfloat16)
        acc = jnp.dot(h.astype(jnp.bfloat16), w2c,
                      preferred_element_type=jnp.float32) + acc
    o_ref[...] = acc.astype(o_ref.dtype)


def kernel(x, w1, b1, w2, b2):
    batch, seq, dim = x.shape
    hidden = w1.shape[1]
    M = batch * seq
    x2d = x.reshape(M, dim)

    b1r = b1.reshape(1, hidden).astype(jnp.float32)
    b2r = b2.reshape(1, dim).astype(jnp.float32)

    TM = 512
    Mp = -(-M // (2 * TM)) * (2 * TM)
    if Mp != M:
        x2d = jnp.pad(x2d, ((0, Mp - M), (0, 0)))

    th = 1024 if hidden % 1024 == 0 else hidden
    cost = pl.CostEstimate(
        flops=int(4 * Mp * dim * hidden),
        transcendentals=int(Mp * hidden),
        bytes_accessed=int(4 * Mp * dim * 2 + 2 * (dim * hidden * 4)),
    )

    out2d = pl.pallas_call(
        functools.partial(_ffn_kernel, th=th),
        out_shape=jax.ShapeDtypeStruct((Mp, dim), x.dtype),
        grid=(Mp // TM,),
        in_specs=[
            pl.BlockSpec((TM, dim), lambda i: (i, 0)),
            pl.BlockSpec((dim, hidden), lambda i: (0, 0)),
            pl.BlockSpec((1, hidden), lambda i: (0, 0)),
            pl.BlockSpec((hidden, dim), lambda i: (0, 0)),
            pl.BlockSpec((1, dim), lambda i: (0, 0)),
        ],
        out_specs=pl.BlockSpec((TM, dim), lambda i: (i, 0)),
        compiler_params=pltpu.CompilerParams(
            dimension_semantics=("parallel",),
            vmem_limit_bytes=61 * 1024 * 1024,
        ),
        cost_estimate=cost,
    )(x2d, w1, b1r, w2, b2r)

    if Mp != M:
        out2d = out2d[:M]
    return out2d.reshape(batch, seq, dim)


# speedup vs baseline: 1.1645x; 1.0064x over previous
import functools
import math

import jax
import jax.numpy as jnp
from jax import lax
from jax.experimental import pallas as pl
from jax.experimental.pallas import tpu as pltpu

_INV_SQRT2 = 1.0 / math.sqrt(2.0)


def _gelu_exact(x):
    return 0.5 * x * (1.0 + lax.erf(x * _INV_SQRT2))


def _ffn_kernel(x_ref, w1_ref, b1_ref, w2_ref, b2_ref, o_ref, *, th):
    xb = x_ref[...].astype(jnp.bfloat16)
    n_h = w1_ref.shape[1] // th
    acc = jnp.broadcast_to(b2_ref[...].astype(jnp.float32), o_ref.shape)
    for c in range(n_h):
        w1c = w1_ref[:, c * th:(c + 1) * th].astype(jnp.bfloat16)
        h = jnp.dot(xb, w1c, preferred_element_type=jnp.float32)
        h = _gelu_exact(h + b1_ref[:, c * th:(c + 1) * th].astype(jnp.float32))
        w2c = w2_ref[c * th:(c + 1) * th, :].astype(jnp.bfloat16)
        acc = jnp.dot(h.astype(jnp.bfloat16), w2c,
                      preferred_element_type=jnp.float32) + acc
    o_ref[...] = acc.astype(o_ref.dtype)


def kernel(x, w1, b1, w2, b2):
    batch, seq, dim = x.shape
    hidden = w1.shape[1]
    M = batch * seq
    x2d = x.reshape(M, dim)

    b1r = b1.reshape(1, hidden).astype(jnp.float32)
    b2r = b2.reshape(1, dim).astype(jnp.float32)

    TM = 1024
    Mp = -(-M // (2 * TM)) * (2 * TM)
    if Mp != M:
        x2d = jnp.pad(x2d, ((0, Mp - M), (0, 0)))

    th = 1024 if hidden % 1024 == 0 else hidden
    cost = pl.CostEstimate(
        flops=int(4 * Mp * dim * hidden),
        transcendentals=int(Mp * hidden),
        bytes_accessed=int(4 * Mp * dim * 2 + 2 * (dim * hidden * 4)),
    )

    out2d = pl.pallas_call(
        functools.partial(_ffn_kernel, th=th),
        out_shape=jax.ShapeDtypeStruct((Mp, dim), x.dtype),
        grid=(Mp // TM,),
        in_specs=[
            pl.BlockSpec((TM, dim), lambda i: (i, 0)),
            pl.BlockSpec((dim, hidden), lambda i: (0, 0)),
            pl.BlockSpec((1, hidden), lambda i: (0, 0)),
            pl.BlockSpec((hidden, dim), lambda i: (0, 0)),
            pl.BlockSpec((1, dim), lambda i: (0, 0)),
        ],
        out_specs=pl.BlockSpec((TM, dim), lambda i: (i, 0)),
        compiler_params=pltpu.CompilerParams(
            dimension_semantics=("parallel",),
            vmem_limit_bytes=61 * 1024 * 1024,
        ),
        cost_estimate=cost,
    )(x2d, w1, b1r, w2, b2r)

    if Mp != M:
        out2d = out2d[:M]
    return out2d.reshape(batch, seq, dim)
